# baseline (speedup 1.0000x reference)
import jax
import jax.numpy as jnp
from jax import lax
from jax.experimental import pallas as pl
from jax.experimental.pallas import tpu as pltpu
from jax.experimental.pallas import tpu_sc as plsc

_NF = 26
_FD = 16
_NPAIR = (_NF * (_NF - 1)) // 2
_ROW_IN = _NF * _FD
_ROW_OUT = _NPAIR * 3 * _FD
_BATCH = 4096
_NW = 32
_LANES = _BATCH // _NW
_NSUB = _LANES // 16


def _pin_body(xt_hbm, out_hbm, xblk, obuf, sem0, sem1, sem2, sem3):
    wid = lax.axis_index("s") * 2 + lax.axis_index("c")
    lane0 = wid * _LANES

    pltpu.sync_copy(xt_hbm.at[:, pl.ds(lane0, _LANES)], xblk)

    sems = (sem0, sem1, sem2, sem3)

    def seg(i, carry):
        seg_base = (i * (2 * _NF - 1 - i)) // 2

        def pairj(j, carry2):
            p = seg_base + (j - i - 1)
            slot = lax.rem(p, 4)
            ir = _FD * i
            jr = _FD * j

            for k in range(4):
                @pl.when((slot == k) & (p >= 4))
                def _drain(k=k):
                    pltpu.make_async_copy(
                        obuf.at[k],
                        out_hbm.at[pl.ds(0, 3 * _FD), pl.ds(lane0, _LANES)],
                        sems[k]).wait()

            for c in range(_FD):
                for u in range(_NSUB):
                    sl = pl.ds(16 * u, 16)
                    av = xblk[ir + c, sl]
                    bv = xblk[jr + c, sl]
                    obuf[slot, c, sl] = av
                    obuf[slot, _FD + c, sl] = bv
                    obuf[slot, 2 * _FD + c, sl] = av * bv

            for k in range(4):
                @pl.when(slot == k)
                def _issue(k=k):
                    pltpu.async_copy(
                        obuf.at[k],
                        out_hbm.at[pl.ds(3 * _FD * p, 3 * _FD),
                                   pl.ds(lane0, _LANES)],
                        sems[k])
            return carry2

        return lax.fori_loop(i + 1, _NF, pairj, carry)

    lax.fori_loop(0, _NF - 1, seg, 0)

    for k in range(4):
        pltpu.make_async_copy(
            obuf.at[k],
            out_hbm.at[pl.ds(0, 3 * _FD), pl.ds(lane0, _LANES)],
            sems[k]).wait()


@jax.jit
def kernel(x):
    xt = x.transpose(1, 2, 0).reshape(_ROW_IN, _BATCH)
    run = pl.kernel(
        _pin_body,
        out_type=jax.ShapeDtypeStruct((_ROW_OUT, _BATCH), jnp.float32),
        scratch_types=[
            pltpu.VMEM((_ROW_IN, _LANES), jnp.float32),
            pltpu.VMEM((4, 3 * _FD, _LANES), jnp.float32),
            pltpu.SemaphoreType.DMA,
            pltpu.SemaphoreType.DMA,
            pltpu.SemaphoreType.DMA,
            pltpu.SemaphoreType.DMA,
        ],
        mesh=plsc.VectorSubcoreMesh(core_axis_name="c", subcore_axis_name="s"),
    )
    out_t = run(xt)
    return out_t.reshape(_NPAIR, 3 * _FD, _BATCH).transpose(2, 0, 1)


# speedup vs baseline: 1.0054x; 1.0054x over previous
import jax
import jax.numpy as jnp
from jax import lax
from jax.experimental import pallas as pl
from jax.experimental.pallas import tpu as pltpu
from jax.experimental.pallas import tpu_sc as plsc

_NF = 26
_FD = 16
_NPAIR = (_NF * (_NF - 1)) // 2
_ROW_IN = _NF * _FD
_ROW_OUT = _NPAIR * 3 * _FD
_BATCH = 4096
_NLG = 16
_NPG = 2
_LANES = _BATCH // _NLG
_NSUB = _LANES // 16
_PSPLIT = 163
_I1, _J1 = 7, 17


def _advance(i, j):
    nj = j + 1
    wrap = nj >= _NF
    ni = lax.select(wrap, i + 1, i)
    nj = lax.select(wrap, ni + 1, nj)
    return ni, nj


def _pin_body(xt_hbm, out_hbm, xblk, obufa, obufb, sa0, sa1, sb0, sb1):
    wid = lax.axis_index("s") * 2 + lax.axis_index("c")
    lg = lax.rem(wid, _NLG)
    pg = wid // _NLG
    lane0 = lg * _LANES

    pltpu.sync_copy(xt_hbm.at[:, pl.ds(lane0, _LANES)], xblk)

    semsa = (sa0, sa1)
    semsb = (sb0, sb1)

    p0 = lax.select(pg == 0, 0, _PSPLIT)
    p1 = lax.select(pg == 0, _PSPLIT, _NPAIR)
    ij0 = (lax.select(pg == 0, 0, _I1), lax.select(pg == 0, 1, _J1))

    def pairp(p, carry):
        i, j = carry
        slot = lax.rem(p, 2)
        ir = _FD * i
        jr = _FD * j

        for k in range(2):
            @pl.when((slot == k) & (p >= p0 + 2))
            def _drain(k=k):
                pltpu.make_async_copy(
                    obufa.at[k],
                    out_hbm.at[pl.ds(0, 24), pl.ds(lane0, _LANES)],
                    semsa[k]).wait()
                pltpu.make_async_copy(
                    obufb.at[k],
                    out_hbm.at[pl.ds(0, 24), pl.ds(lane0, _LANES)],
                    semsb[k]).wait()

        for c in range(_FD):
            for u in range(_NSUB):
                sl = pl.ds(16 * u, 16)
                av = xblk[ir + c, sl]
                bv = xblk[jr + c, sl]
                obufa[slot, c, sl] = av
                if c < 8:
                    obufa[slot, _FD + c, sl] = bv
                else:
                    obufb[slot, c - 8, sl] = bv
                obufb[slot, 8 + c, sl] = av * bv

        for k in range(2):
            @pl.when(slot == k)
            def _issue(k=k):
                pltpu.async_copy(
                    obufa.at[k],
                    out_hbm.at[pl.ds(3 * _FD * p, 24), pl.ds(lane0, _LANES)],
                    semsa[k])
                pltpu.async_copy(
                    obufb.at[k],
                    out_hbm.at[pl.ds(3 * _FD * p + 24, 24),
                               pl.ds(lane0, _LANES)],
                    semsb[k])
        return _advance(i, j)

    lax.fori_loop(p0, p1, pairp, ij0)

    for k in range(2):
        pltpu.make_async_copy(
            obufa.at[k],
            out_hbm.at[pl.ds(0, 24), pl.ds(lane0, _LANES)],
            semsa[k]).wait()
        pltpu.make_async_copy(
            obufb.at[k],
            out_hbm.at[pl.ds(0, 24), pl.ds(lane0, _LANES)],
            semsb[k]).wait()


@jax.jit
def kernel(x):
    xt = x.transpose(1, 2, 0).reshape(_ROW_IN, _BATCH)
    run = pl.kernel(
        _pin_body,
        out_type=jax.ShapeDtypeStruct((_ROW_OUT, _BATCH), jnp.float32),
        scratch_types=[
            pltpu.VMEM((_ROW_IN, _LANES), jnp.float32),
            pltpu.VMEM((2, 24, _LANES), jnp.float32),
            pltpu.VMEM((2, 24, _LANES), jnp.float32),
            pltpu.SemaphoreType.DMA,
            pltpu.SemaphoreType.DMA,
            pltpu.SemaphoreType.DMA,
            pltpu.SemaphoreType.DMA,
        ],
        mesh=plsc.VectorSubcoreMesh(core_axis_name="c", subcore_axis_name="s"),
    )
    out_t = run(xt)
    return out_t.reshape(_NPAIR, 3 * _FD, _BATCH).transpose(2, 0, 1)


# speedup vs baseline: 1.0080x; 1.0026x over previous
import jax
import jax.numpy as jnp
from jax import lax
from jax.experimental import pallas as pl
from jax.experimental.pallas import tpu as pltpu
from jax.experimental.pallas import tpu_sc as plsc

_NF = 26
_FD = 16
_NPAIR = (_NF * (_NF - 1)) // 2
_ROW_IN = _NF * _FD
_ROW_OUT = _NPAIR * 3 * _FD
_BATCH = 4096
_NLG = 16
_LANES = _BATCH // _NLG
_NSUB = _LANES // 16
_PSPLIT = 163
_I1, _J1 = 7, 17


def _advance(i, j):
    nj = j + 1
    wrap = nj >= _NF
    ni = lax.select(wrap, i + 1, i)
    nj = lax.select(wrap, ni + 1, nj)
    return ni, nj


def _pin_body(xt_hbm, out_hbm, xblk, obuf, semc, sp0, sp1):
    wid = lax.axis_index("s") * 2 + lax.axis_index("c")
    lg = lax.rem(wid, _NLG)
    pg = wid // _NLG
    lane0 = lg * _LANES

    pltpu.sync_copy(xt_hbm.at[:, pl.ds(lane0, _LANES)], xblk)

    semsp = (sp0, sp1)

    p0 = lax.select(pg == 0, 0, _PSPLIT)
    p1 = lax.select(pg == 0, _PSPLIT, _NPAIR)
    ij0 = (lax.select(pg == 0, 0, _I1), lax.select(pg == 0, 1, _J1))

    def pairp(p, carry):
        i, j = carry
        slot = lax.rem(p, 2)
        ir = _FD * i
        jr = _FD * j
        r0 = 3 * _FD * p

        @pl.when(p >= p0 + 2)
        def _drainc():
            for _ in range(2):
                pltpu.make_async_copy(
                    xblk.at[pl.ds(0, _FD), :],
                    out_hbm.at[pl.ds(0, _FD), pl.ds(lane0, _LANES)],
                    semc).wait()

        pltpu.async_copy(
            xblk.at[pl.ds(ir, _FD), :],
            out_hbm.at[pl.ds(r0, _FD), pl.ds(lane0, _LANES)],
            semc)
        pltpu.async_copy(
            xblk.at[pl.ds(jr, _FD), :],
            out_hbm.at[pl.ds(r0 + _FD, _FD), pl.ds(lane0, _LANES)],
            semc)

        for k in range(2):
            @pl.when((slot == k) & (p >= p0 + 2))
            def _drainp(k=k):
                pltpu.make_async_copy(
                    obuf.at[k],
                    out_hbm.at[pl.ds(0, _FD), pl.ds(lane0, _LANES)],
                    semsp[k]).wait()

        for c in range(_FD):
            for u in range(_NSUB):
                sl = pl.ds(16 * u, 16)
                obuf[slot, c, sl] = xblk[ir + c, sl] * xblk[jr + c, sl]

        for k in range(2):
            @pl.when(slot == k)
            def _issuep(k=k):
                pltpu.async_copy(
                    obuf.at[k],
                    out_hbm.at[pl.ds(r0 + 2 * _FD, _FD), pl.ds(lane0, _LANES)],
                    semsp[k])
        return _advance(i, j)

    lax.fori_loop(p0, p1, pairp, ij0)

    for _ in range(4):
        pltpu.make_async_copy(
            xblk.at[pl.ds(0, _FD), :],
            out_hbm.at[pl.ds(0, _FD), pl.ds(lane0, _LANES)],
            semc).wait()
    for k in range(2):
        pltpu.make_async_copy(
            obuf.at[k],
            out_hbm.at[pl.ds(0, _FD), pl.ds(lane0, _LANES)],
            semsp[k]).wait()


@jax.jit
def kernel(x):
    xt = x.transpose(1, 2, 0).reshape(_ROW_IN, _BATCH)
    run = pl.kernel(
        _pin_body,
        out_type=jax.ShapeDtypeStruct((_ROW_OUT, _BATCH), jnp.float32),
        scratch_types=[
            pltpu.VMEM((_ROW_IN, _LANES), jnp.float32),
            pltpu.VMEM((2, _FD, _LANES), jnp.float32),
            pltpu.SemaphoreType.DMA,
            pltpu.SemaphoreType.DMA,
            pltpu.SemaphoreType.DMA,
        ],
        mesh=plsc.VectorSubcoreMesh(core_axis_name="c", subcore_axis_name="s"),
    )
    out_t = run(xt)
    return out_t.reshape(_NPAIR, 3 * _FD, _BATCH).transpose(2, 0, 1)


# speedup vs baseline: 2.3313x; 2.3128x over previous
import jax
import jax.numpy as jnp
from jax import lax
from jax.experimental import pallas as pl
from jax.experimental.pallas import tpu as pltpu
from jax.experimental.pallas import tpu_sc as plsc

_NF = 26
_FD = 16
_NPAIR = (_NF * (_NF - 1)) // 2
_ROW_IN = _NF * _FD
_ROW_OUT = _NPAIR * 3 * _FD
_BATCH = 4096
_NLG = 16
_LANES = _BATCH // _NLG
_NSUB = _LANES // 16
_PSPLIT = 163
_I1, _J1 = 7, 17


def _advance(i, j):
    nj = j + 1
    wrap = nj >= _NF
    ni = lax.select(wrap, i + 1, i)
    nj = lax.select(wrap, ni + 1, nj)
    return ni, nj


def _pin_body(xt_hbm, out_hbm, xblk, obuf, semc, sp0, sp1):
    wid = lax.axis_index("s") * 2 + lax.axis_index("c")
    lg = lax.rem(wid, _NLG)
    pg = wid // _NLG
    lane0 = lg * _LANES

    pltpu.sync_copy(xt_hbm.at[:, pl.ds(lane0, _LANES)], xblk)

    semsp = (sp0, sp1)

    p0 = lax.select(pg == 0, 0, _PSPLIT)
    p1 = lax.select(pg == 0, _PSPLIT, _NPAIR)
    ij0 = (lax.select(pg == 0, 0, _I1), lax.select(pg == 0, 1, _J1))

    def pairp(p, carry):
        i, j = carry
        slot = lax.rem(p, 2)
        ir = _FD * i
        jr = _FD * j
        r0 = 3 * _FD * p

        @pl.when(p >= p0 + 2)
        def _drainc():
            for _ in range(2):
                pltpu.make_async_copy(
                    xblk.at[pl.ds(0, _FD), :],
                    out_hbm.at[pl.ds(0, _FD), pl.ds(lane0, _LANES)],
                    semc).wait()

        pltpu.async_copy(
            xblk.at[pl.ds(ir, _FD), :],
            out_hbm.at[pl.ds(r0, _FD), pl.ds(lane0, _LANES)],
            semc)
        pltpu.async_copy(
            xblk.at[pl.ds(jr, _FD), :],
            out_hbm.at[pl.ds(r0 + _FD, _FD), pl.ds(lane0, _LANES)],
            semc)

        for k in range(2):
            @pl.when((slot == k) & (p >= p0 + 2))
            def _drainp(k=k):
                pltpu.make_async_copy(
                    obuf.at[k],
                    out_hbm.at[pl.ds(0, _FD), pl.ds(lane0, _LANES)],
                    semsp[k]).wait()

        chunks = [(c, u) for c in range(_FD) for u in range(_NSUB)]
        for g in range(0, len(chunks), 8):
            grp = chunks[g:g + 8]
            avs = [xblk[ir + c, pl.ds(16 * u, 16)] for (c, u) in grp]
            bvs = [xblk[jr + c, pl.ds(16 * u, 16)] for (c, u) in grp]
            prods = [a * b for a, b in zip(avs, bvs)]
            for (c, u), pr in zip(grp, prods):
                obuf[slot, c, pl.ds(16 * u, 16)] = pr

        for k in range(2):
            @pl.when(slot == k)
            def _issuep(k=k):
                pltpu.async_copy(
                    obuf.at[k],
                    out_hbm.at[pl.ds(r0 + 2 * _FD, _FD), pl.ds(lane0, _LANES)],
                    semsp[k])
        return _advance(i, j)

    lax.fori_loop(p0, p1, pairp, ij0)

    for _ in range(4):
        pltpu.make_async_copy(
            xblk.at[pl.ds(0, _FD), :],
            out_hbm.at[pl.ds(0, _FD), pl.ds(lane0, _LANES)],
            semc).wait()
    for k in range(2):
        pltpu.make_async_copy(
            obuf.at[k],
            out_hbm.at[pl.ds(0, _FD), pl.ds(lane0, _LANES)],
            semsp[k]).wait()


@jax.jit
def kernel(x):
    xt = x.transpose(1, 2, 0).reshape(_ROW_IN, _BATCH)
    run = pl.kernel(
        _pin_body,
        out_type=jax.ShapeDtypeStruct((_ROW_OUT, _BATCH), jnp.float32),
        scratch_types=[
            pltpu.VMEM((_ROW_IN, _LANES), jnp.float32),
            pltpu.VMEM((2, _FD, _LANES), jnp.float32),
            pltpu.SemaphoreType.DMA,
            pltpu.SemaphoreType.DMA,
            pltpu.SemaphoreType.DMA,
        ],
        mesh=plsc.VectorSubcoreMesh(core_axis_name="c", subcore_axis_name="s"),
    )
    out_t = run(xt)
    return out_t.reshape(_NPAIR, 3 * _FD, _BATCH).transpose(2, 0, 1)
